# baseline (speedup 1.0000x reference)
import jax
import jax.numpy as jnp
from jax import lax
from jax.experimental import pallas as pl
from jax.experimental.pallas import tpu as pltpu
from jax.experimental.pallas import tpu_sc as plsc

B = 1024
V = 100000
VP = 100352
NT = 16
CHUNKS = 49
CW = 128
TILE_N = CHUNKS * CW
BLK = 3584
GRID = VP // BLK


def _colstats_body(x_ref, s1_ref, s2_ref):
    x = x_ref[...]
    s1 = jnp.sum(x, axis=0, keepdims=True)
    s2 = jnp.sum(x * x, axis=0, keepdims=True)
    col = pl.program_id(0) * BLK + lax.broadcasted_iota(jnp.int32, (1, BLK), 1)
    valid = col < V
    s1 = jnp.where(valid, s1, 0.0)
    s2 = jnp.where(valid, s2, 0.0)
    s1_ref[...] = jnp.broadcast_to(s1, (8, BLK))
    s2_ref[...] = jnp.broadcast_to(s2, (8, BLK))


def _colstats(x):
    return pl.pallas_call(
        _colstats_body,
        grid=(GRID,),
        in_specs=[pl.BlockSpec((B, BLK), lambda i: (0, i))],
        out_specs=[
            pl.BlockSpec((8, BLK), lambda i: (0, i)),
            pl.BlockSpec((8, BLK), lambda i: (0, i)),
        ],
        out_shape=[
            jax.ShapeDtypeStruct((8, VP), jnp.float32),
            jax.ShapeDtypeStruct((8, VP), jnp.float32),
        ],
    )(x)


def _sc_body(s1_hbm, s2_hbm, tgt_hbm, z_hbm, out_hbm,
             idx_v, g1_v, g2_v, m_v, ones_v, acc_v, mask_sh, sem):
    wid = lax.axis_index("s")
    base = wid * TILE_N

    pltpu.sync_copy(tgt_hbm.at[wid], idx_v)
    pltpu.sync_copy(z_hbm.at[wid], mask_sh.at[pl.ds(base, TILE_N)])
    for i in range(CW // 16):
        ones_v[pl.ds(i * 16, 16)] = jnp.ones((16,), jnp.float32)
    plsc.subcore_barrier()

    def scat(k, c):
        pltpu.sync_copy(ones_v, mask_sh.at[idx_v.at[k]])
        return c
    lax.fori_loop(0, CHUNKS, scat, 0)
    plsc.subcore_barrier()

    def gat(k, acc):
        pltpu.async_copy(s1_hbm.at[idx_v.at[k]], g1_v, sem).wait()
        pltpu.async_copy(s2_hbm.at[idx_v.at[k]], g2_v, sem).wait()
        pltpu.sync_copy(mask_sh.at[pl.ds(base + k * CW, CW)], m_v)

        def inner(i, a):
            g1 = g1_v[pl.ds(i * 16, 16)]
            g2 = g2_v[pl.ds(i * 16, 16)]
            l = 2.0 * m_v[pl.ds(i * 16, 16)] - 1.0
            return a + (g2 - 2.0 * l * g1)
        return lax.fori_loop(0, CW // 16, inner, acc)

    acc = lax.fori_loop(0, CHUNKS, gat, jnp.zeros((16,), jnp.float32))
    acc_v[...] = acc
    pltpu.sync_copy(acc_v, out_hbm.at[wid])


_sc_loss = pl.kernel(
    _sc_body,
    out_type=jax.ShapeDtypeStruct((NT, 16), jnp.float32),
    mesh=plsc.VectorSubcoreMesh(
        core_axis_name="c", subcore_axis_name="s", num_cores=1),
    scratch_types=[
        pltpu.VMEM((CHUNKS, CW), jnp.int32),
        pltpu.VMEM((CW,), jnp.float32),
        pltpu.VMEM((CW,), jnp.float32),
        pltpu.VMEM((CW,), jnp.float32),
        pltpu.VMEM((CW,), jnp.float32),
        pltpu.VMEM((16,), jnp.float32),
        pltpu.VMEM_SHARED((VP,), jnp.float32),
        pltpu.SemaphoreType.DMA,
    ],
)


@jax.jit
def kernel(inputs, targets):
    s1_8, s2_8 = _colstats(inputs)
    s1 = s1_8[0]
    s2 = s2_8[0]
    tgt = jnp.concatenate(
        [targets.astype(jnp.int32),
         jnp.full((VP - V,), V, jnp.int32)]).reshape(NT, CHUNKS, CW)
    zeros = jnp.zeros((NT, TILE_N), jnp.float32)
    partials = _sc_loss(s1, s2, tgt, zeros)
    return jnp.sum(partials) / (B * V) + 1.0


# speedup vs baseline: 1.0725x; 1.0725x over previous
import jax
import jax.numpy as jnp
from jax import lax
from jax.experimental import pallas as pl
from jax.experimental.pallas import tpu as pltpu
from jax.experimental.pallas import tpu_sc as plsc

B = 1024
V = 100000
VP = 100352
NT = 16
CHUNKS = 49
CW = 128
TILE_N = CHUNKS * CW

WT = 1536
W_SC = 32 * WT
C_TC = V - W_SC
BLK = 2048
TC_BLK0 = W_SC // BLK
GRID = -(-C_TC // BLK)
C_TCP = GRID * BLK

NR = 32
NGROUPS = B // NR


def _colstats_body(x_ref, s1_ref, s2_ref):
    x = x_ref[...]
    s1_ref[...] = jnp.broadcast_to(jnp.sum(x, axis=0, keepdims=True), (8, BLK))
    s2_ref[...] = jnp.broadcast_to(jnp.sum(x * x, axis=0, keepdims=True),
                                   (8, BLK))


def _colstats(x):
    return pl.pallas_call(
        _colstats_body,
        grid=(GRID,),
        in_specs=[pl.BlockSpec((B, BLK), lambda i: (0, TC_BLK0 + i))],
        out_specs=[
            pl.BlockSpec((8, BLK), lambda i: (0, i)),
            pl.BlockSpec((8, BLK), lambda i: (0, i)),
        ],
        out_shape=[
            jax.ShapeDtypeStruct((8, C_TCP), jnp.float32),
            jax.ShapeDtypeStruct((8, C_TCP), jnp.float32),
        ],
    )(x)


def _sc_colstats_body(x_hbm, s1_hbm, s2_hbm,
                      buf0, buf1, a1_v, a2_v, sem0, sem1):
    wid = lax.axis_index("s") * 2 + lax.axis_index("c")
    c0 = wid * WT

    def zero(i, c):
        z = jnp.zeros((16,), jnp.float32)
        a1_v[pl.ds(i * 16, 16)] = z
        a2_v[pl.ds(i * 16, 16)] = z
        return c
    lax.fori_loop(0, WT // 16, zero, 0)

    def fire(g, buf, sem):
        return pltpu.async_copy(
            x_hbm.at[pl.ds(g * NR, NR), pl.ds(c0, WT)], buf, sem)

    def accumulate(buf):
        def col(i, c):
            a1 = a1_v[pl.ds(i * 16, 16)]
            a2 = a2_v[pl.ds(i * 16, 16)]
            for r in range(NR):
                x = buf[r, pl.ds(i * 16, 16)]
                a1 = a1 + x
                a2 = a2 + x * x
            a1_v[pl.ds(i * 16, 16)] = a1
            a2_v[pl.ds(i * 16, 16)] = a2
            return c
        lax.fori_loop(0, WT // 16, col, 0)

    fire(0, buf0, sem0)
    fire(1, buf1, sem1)

    def body(g2, c):
        pltpu.make_async_copy(
            x_hbm.at[pl.ds(0, NR), pl.ds(c0, WT)], buf0, sem0).wait()
        accumulate(buf0)

        @pl.when(2 * g2 + 2 < NGROUPS)
        def _():
            fire(2 * g2 + 2, buf0, sem0)

        pltpu.make_async_copy(
            x_hbm.at[pl.ds(0, NR), pl.ds(c0, WT)], buf1, sem1).wait()
        accumulate(buf1)

        @pl.when(2 * g2 + 3 < NGROUPS)
        def _():
            fire(2 * g2 + 3, buf1, sem1)
        return c
    lax.fori_loop(0, NGROUPS // 2, body, 0)

    pltpu.sync_copy(a1_v, s1_hbm.at[pl.ds(wid * WT, WT)])
    pltpu.sync_copy(a2_v, s2_hbm.at[pl.ds(wid * WT, WT)])


_sc_colstats = pl.kernel(
    _sc_colstats_body,
    out_type=[
        jax.ShapeDtypeStruct((W_SC,), jnp.float32),
        jax.ShapeDtypeStruct((W_SC,), jnp.float32),
    ],
    mesh=plsc.VectorSubcoreMesh(
        core_axis_name="c", subcore_axis_name="s", num_cores=2),
    scratch_types=[
        pltpu.VMEM((NR, WT), jnp.float32),
        pltpu.VMEM((NR, WT), jnp.float32),
        pltpu.VMEM((WT,), jnp.float32),
        pltpu.VMEM((WT,), jnp.float32),
        pltpu.SemaphoreType.DMA,
        pltpu.SemaphoreType.DMA,
    ],
)


GK = 7


def _sc_loss_body(s1_hbm, s2_hbm, tgt_hbm, z_hbm, out_hbm,
                  idx_v, g1_v, g2_v, m_v, ones_v, acc_v, mask_sh, sem):
    wid = lax.axis_index("s")
    base = wid * TILE_N

    pltpu.sync_copy(tgt_hbm.at[wid], idx_v)
    pltpu.sync_copy(z_hbm.at[wid], mask_sh.at[pl.ds(base, TILE_N)])
    for i in range(CW // 16):
        ones_v[pl.ds(i * 16, 16)] = jnp.ones((16,), jnp.float32)
    plsc.subcore_barrier()

    def scat(kk, c):
        def f(j, c2):
            pltpu.async_copy(ones_v, mask_sh.at[idx_v.at[kk * GK + j]], sem)
            return c2
        lax.fori_loop(0, GK, f, 0)

        def d(j, c2):
            pltpu.make_async_copy(
                ones_v, mask_sh.at[idx_v.at[kk * GK + j]], sem).wait()
            return c2
        lax.fori_loop(0, GK, d, 0)
        return c
    lax.fori_loop(0, CHUNKS // GK, scat, 0)
    plsc.subcore_barrier()

    def gat(kk, acc):
        def f(j, c2):
            k = kk * GK + j
            pltpu.async_copy(s1_hbm.at[idx_v.at[k]], g1_v.at[k], sem)
            pltpu.async_copy(s2_hbm.at[idx_v.at[k]], g2_v.at[k], sem)
            return c2
        lax.fori_loop(0, GK, f, 0)

        def d(j, acc2):
            k = kk * GK + j
            pltpu.make_async_copy(s1_hbm.at[idx_v.at[k]], g1_v.at[k],
                                  sem).wait()
            pltpu.make_async_copy(s2_hbm.at[idx_v.at[k]], g2_v.at[k],
                                  sem).wait()

            def inner(i, a):
                g1 = g1_v[k, pl.ds(i * 16, 16)]
                g2 = g2_v[k, pl.ds(i * 16, 16)]
                l = 2.0 * m_v[pl.ds(k * CW + i * 16, 16)] - 1.0
                return a + (g2 - 2.0 * l * g1)
            return lax.fori_loop(0, CW // 16, inner, acc2)
        return lax.fori_loop(0, GK, d, acc)

    pltpu.sync_copy(mask_sh.at[pl.ds(base, TILE_N)], m_v)
    acc = lax.fori_loop(0, CHUNKS // GK, gat, jnp.zeros((16,), jnp.float32))
    acc_v[...] = acc
    pltpu.sync_copy(acc_v, out_hbm.at[wid])


_sc_loss = pl.kernel(
    _sc_loss_body,
    out_type=jax.ShapeDtypeStruct((NT, 16), jnp.float32),
    mesh=plsc.VectorSubcoreMesh(
        core_axis_name="c", subcore_axis_name="s", num_cores=1),
    scratch_types=[
        pltpu.VMEM((CHUNKS, CW), jnp.int32),
        pltpu.VMEM((CHUNKS, CW), jnp.float32),
        pltpu.VMEM((CHUNKS, CW), jnp.float32),
        pltpu.VMEM((TILE_N,), jnp.float32),
        pltpu.VMEM((CW,), jnp.float32),
        pltpu.VMEM((16,), jnp.float32),
        pltpu.VMEM_SHARED((VP,), jnp.float32),
        pltpu.SemaphoreType.DMA,
    ],
)


@jax.jit
def kernel(inputs, targets):
    s1_tc8, s2_tc8 = _colstats(inputs)
    s1_sc, s2_sc = _sc_colstats(inputs)
    pad = jnp.zeros((VP - V,), jnp.float32)
    s1 = jnp.concatenate([s1_sc, s1_tc8[0, :C_TC], pad])
    s2 = jnp.concatenate([s2_sc, s2_tc8[0, :C_TC], pad])
    tgt = jnp.concatenate(
        [targets.astype(jnp.int32),
         jnp.full((VP - V,), V, jnp.int32)]).reshape(NT, CHUNKS, CW)
    zeros = jnp.zeros((NT, TILE_N), jnp.float32)
    partials = _sc_loss(s1, s2, tgt, zeros)
    return jnp.sum(partials) / (B * V) + 1.0
